# baseline (speedup 1.0000x reference)
import functools

import jax
import jax.numpy as jnp
from jax.experimental import pallas as pl
from jax.experimental.pallas import tpu as pltpu

_LN_EPS = 1e-5
_NORM_EPS = 1e-12


def _cbnorm_body(cb_ref, out_ref):
    cb = cb_ref[0]
    norm = jnp.sqrt(jnp.sum(cb * cb, axis=1, keepdims=True))
    out_ref[0] = (cb / (norm + _NORM_EPS)).astype(jnp.bfloat16)


def _vq_body(x_ref, rp_ref, cbn_ref, out_ref, xn_ref, q_ref, bv_ref,
             bb_ref, *, kt_num, tile_k, k_total):
    h = pl.program_id(1)
    kt = pl.program_id(2)
    tile_n = bv_ref.shape[0]
    n_blk = tile_k // 128

    @pl.when(jnp.logical_and(h == 0, kt == 0))
    def _layernorm():
        xt = x_ref[...]
        mean = jnp.mean(xt, axis=1, keepdims=True)
        cent = xt - mean
        var = jnp.mean(cent * cent, axis=1, keepdims=True)
        xn_ref[...] = (cent / jnp.sqrt(var + _LN_EPS)).astype(jnp.bfloat16)

    @pl.when(kt == 0)
    def _project():
        proj = jax.lax.dot_general(
            xn_ref[...], rp_ref[0].astype(jnp.bfloat16),
            (((1,), (0,)), ((), ())),
            preferred_element_type=jnp.float32)
        norm = jnp.sqrt(jnp.sum(proj * proj, axis=1, keepdims=True))
        q_ref[...] = (proj / (norm + _NORM_EPS)).astype(jnp.bfloat16)

    sim = jax.lax.dot_general(
        q_ref[...], cbn_ref[0], (((1,), (1,)), ((), ())),
        preferred_element_type=jnp.float32)

    @pl.when(kt == 0)
    def _init():
        bv = sim[:, 0:128]
        bb = jnp.zeros((tile_n, 128), jnp.int32)
        for cc in range(1, n_blk):
            seg = sim[:, cc * 128:(cc + 1) * 128]
            mask = seg > bv
            bv = jnp.where(mask, seg, bv)
            bb = jnp.where(mask, cc, bb)
        bv_ref[...] = bv
        bb_ref[...] = bb

    @pl.when(kt > 0)
    def _update():
        bv = bv_ref[...]
        bb = bb_ref[...]
        for cc in range(n_blk):
            seg = sim[:, cc * 128:(cc + 1) * 128]
            mask = seg > bv
            bv = jnp.where(mask, seg, bv)
            bb = jnp.where(mask, kt * n_blk + cc, bb)
        bv_ref[...] = bv
        bb_ref[...] = bb

    @pl.when(kt == kt_num - 1)
    def _finish():
        bv = bv_ref[...]
        k_idx = bb_ref[...] * 128 + jax.lax.broadcasted_iota(
            jnp.int32, (tile_n, 128), 1)
        m = jnp.max(bv, axis=1, keepdims=True)
        winners = jnp.where(bv == m, k_idx, k_total)
        out_ref[0, 0] = jnp.min(winners, axis=1, keepdims=True)


def kernel(x, rand_projs, codebook):
    b, n, d = x.shape
    h, _, e = rand_projs.shape
    _, k, _ = codebook.shape
    rows = b * n

    tile_n = min(1024, rows)
    tile_k = min(1024, k)
    nt = rows // tile_n
    kt_num = k // tile_k

    x2 = x.reshape(rows, d)

    cbn = pl.pallas_call(
        _cbnorm_body,
        grid=(h, kt_num),
        in_specs=[pl.BlockSpec((1, tile_k, e), lambda j, s: (j, s, 0))],
        out_specs=pl.BlockSpec((1, tile_k, e), lambda j, s: (j, s, 0)),
        out_shape=jax.ShapeDtypeStruct((h, k, e), jnp.bfloat16),
    )(codebook)

    body = functools.partial(_vq_body, kt_num=kt_num, tile_k=tile_k,
                             k_total=k)

    out = pl.pallas_call(
        body,
        grid=(nt, h, kt_num),
        in_specs=[
            pl.BlockSpec((tile_n, d), lambda i, j, s: (i, 0)),
            pl.BlockSpec((1, d, e), lambda i, j, s: (j, 0, 0)),
            pl.BlockSpec((1, tile_k, e), lambda i, j, s: (j, s, 0)),
        ],
        out_specs=pl.BlockSpec((1, 1, tile_n, 1),
                               lambda i, j, s: (i, j, 0, 0)),
        out_shape=jax.ShapeDtypeStruct((nt, h, tile_n, 1), jnp.int32),
        scratch_shapes=[
            pltpu.VMEM((tile_n, d), jnp.bfloat16),
            pltpu.VMEM((tile_n, e), jnp.bfloat16),
            pltpu.VMEM((tile_n, 128), jnp.float32),
            pltpu.VMEM((tile_n, 128), jnp.int32),
        ],
    )(x2, rand_projs, cbn)

    return out.reshape(nt, h, tile_n).transpose(0, 2, 1).reshape(b, n, h)


# speedup vs baseline: 1.5169x; 1.5169x over previous
import functools

import jax
import jax.numpy as jnp
from jax.experimental import pallas as pl
from jax.experimental.pallas import tpu as pltpu

_LN_EPS = 1e-5
_NORM_EPS = 1e-12
_ROW_CHUNK = 64


def _cbnorm_body(cb_ref, out_ref):
    cb = cb_ref[0]
    norm = jnp.sqrt(jnp.sum(cb * cb, axis=1, keepdims=True))
    out_ref[0] = (cb / (norm + _NORM_EPS)).astype(jnp.bfloat16)


def _vq_body(x_ref, rp_ref, cbn_ref, out_ref, xn_ref, q_ref, bv_ref,
             bb_ref, *, kt_num, tile_k, k_total):
    h = pl.program_id(1)
    kt = pl.program_id(2)
    tile_n = bv_ref.shape[0]
    n_blk = tile_k // 128

    @pl.when(jnp.logical_and(h == 0, kt == 0))
    def _layernorm():
        xt = x_ref[...]
        mean = jnp.mean(xt, axis=1, keepdims=True)
        cent = xt - mean
        var = jnp.mean(cent * cent, axis=1, keepdims=True)
        xn_ref[...] = (cent / jnp.sqrt(var + _LN_EPS)).astype(jnp.bfloat16)

    @pl.when(kt == 0)
    def _project():
        proj = jax.lax.dot_general(
            xn_ref[...], rp_ref[0].astype(jnp.bfloat16),
            (((1,), (0,)), ((), ())),
            preferred_element_type=jnp.float32)
        norm = jnp.sqrt(jnp.sum(proj * proj, axis=1, keepdims=True))
        q_ref[...] = (proj / (norm + _NORM_EPS)).astype(jnp.bfloat16)
        bv_ref[...] = jnp.full((tile_n, 128), -jnp.inf, jnp.float32)

    sim = jax.lax.dot_general(
        q_ref[...], cbn_ref[0], (((1,), (1,)), ((), ())),
        preferred_element_type=jnp.float32)

    for r in range(0, tile_n, _ROW_CHUNK):
        bv = bv_ref[r:r + _ROW_CHUNK, :]
        bb = bb_ref[r:r + _ROW_CHUNK, :]
        for cc in range(n_blk):
            seg = sim[r:r + _ROW_CHUNK, cc * 128:(cc + 1) * 128]
            mask = seg > bv
            bv = jnp.where(mask, seg, bv)
            bb = jnp.where(mask, kt * n_blk + cc, bb)
        bv_ref[r:r + _ROW_CHUNK, :] = bv
        bb_ref[r:r + _ROW_CHUNK, :] = bb

    @pl.when(kt == kt_num - 1)
    def _finish():
        bv = bv_ref[...]
        k_idx = bb_ref[...] * 128 + jax.lax.broadcasted_iota(
            jnp.int32, (tile_n, 128), 1)
        m = jnp.max(bv, axis=1, keepdims=True)
        winners = jnp.where(bv == m, k_idx, k_total)
        out_ref[0, 0] = jnp.min(winners, axis=1, keepdims=True)


def kernel(x, rand_projs, codebook):
    b, n, d = x.shape
    h, _, e = rand_projs.shape
    _, k, _ = codebook.shape
    rows = b * n

    tile_n = min(1024, rows)
    tile_k = min(1024, k)
    nt = rows // tile_n
    kt_num = k // tile_k

    x2 = x.reshape(rows, d)

    cbn = pl.pallas_call(
        _cbnorm_body,
        grid=(h, kt_num),
        in_specs=[pl.BlockSpec((1, tile_k, e), lambda j, s: (j, s, 0))],
        out_specs=pl.BlockSpec((1, tile_k, e), lambda j, s: (j, s, 0)),
        out_shape=jax.ShapeDtypeStruct((h, k, e), jnp.bfloat16),
    )(codebook)

    body = functools.partial(_vq_body, kt_num=kt_num, tile_k=tile_k,
                             k_total=k)

    out = pl.pallas_call(
        body,
        grid=(nt, h, kt_num),
        in_specs=[
            pl.BlockSpec((tile_n, d), lambda i, j, s: (i, 0)),
            pl.BlockSpec((1, d, e), lambda i, j, s: (j, 0, 0)),
            pl.BlockSpec((1, tile_k, e), lambda i, j, s: (j, s, 0)),
        ],
        out_specs=pl.BlockSpec((1, 1, tile_n, 1),
                               lambda i, j, s: (i, j, 0, 0)),
        out_shape=jax.ShapeDtypeStruct((nt, h, tile_n, 1), jnp.int32),
        scratch_shapes=[
            pltpu.VMEM((tile_n, d), jnp.bfloat16),
            pltpu.VMEM((tile_n, e), jnp.bfloat16),
            pltpu.VMEM((tile_n, 128), jnp.float32),
            pltpu.VMEM((tile_n, 128), jnp.int32),
        ],
    )(x2, rand_projs, cbn)

    return out.reshape(nt, h, tile_n).transpose(0, 2, 1).reshape(b, n, h)


# speedup vs baseline: 1.6079x; 1.0600x over previous
import functools

import jax
import jax.numpy as jnp
from jax.experimental import pallas as pl
from jax.experimental.pallas import tpu as pltpu

_LN_EPS = 1e-5
_NORM_EPS = 1e-12
_ROW_CHUNK = 64


def _cbnorm_body(cb_ref, out_ref):
    cb = cb_ref[0]
    norm = jnp.sqrt(jnp.sum(cb * cb, axis=1, keepdims=True))
    out_ref[0] = (cb / (norm + _NORM_EPS)).astype(jnp.bfloat16)


def _fold(src, bv_ref, bb_ref, base_blk, tile_n, tile_k):
    n_blk = tile_k // 128
    for r in range(0, tile_n, _ROW_CHUNK):
        bv = bv_ref[r:r + _ROW_CHUNK, :]
        bb = bb_ref[r:r + _ROW_CHUNK, :]
        for cc in range(n_blk):
            seg = src[r:r + _ROW_CHUNK, cc * 128:(cc + 1) * 128]
            mask = seg > bv
            bv = jnp.where(mask, seg, bv)
            bb = jnp.where(mask, base_blk + cc, bb)
        bv_ref[r:r + _ROW_CHUNK, :] = bv
        bb_ref[r:r + _ROW_CHUNK, :] = bb


def _vq_body(x_ref, rp_ref, cbn_ref, out_ref, xn_ref, q_ref, bv_ref,
             bb_ref, simE_ref, simO_ref, *, kt2_num, tile_k, k_total):
    h = pl.program_id(1)
    kt2 = pl.program_id(2)
    tile_n = bv_ref.shape[0]
    n_blk = tile_k // 128

    @pl.when(jnp.logical_and(h == 0, kt2 == 0))
    def _layernorm():
        xt = x_ref[...]
        mean = jnp.mean(xt, axis=1, keepdims=True)
        cent = xt - mean
        var = jnp.mean(cent * cent, axis=1, keepdims=True)
        xn_ref[...] = (cent / jnp.sqrt(var + _LN_EPS)).astype(jnp.bfloat16)

    @pl.when(kt2 == 0)
    def _project():
        proj = jax.lax.dot_general(
            xn_ref[...], rp_ref[0].astype(jnp.bfloat16),
            (((1,), (0,)), ((), ())),
            preferred_element_type=jnp.float32)
        norm = jnp.sqrt(jnp.sum(proj * proj, axis=1, keepdims=True))
        q_ref[...] = (proj / (norm + _NORM_EPS)).astype(jnp.bfloat16)
        bv_ref[...] = jnp.full((tile_n, 128), -jnp.inf, jnp.float32)
        simO_ref[...] = jnp.full((tile_n, tile_k), -jnp.inf, jnp.float32)

    q = q_ref[...]
    cb2 = cbn_ref[0]

    simE_ref[...] = jax.lax.dot_general(
        q, cb2[0:tile_k, :], (((1,), (1,)), ((), ())),
        preferred_element_type=jnp.float32)
    _fold(simO_ref, bv_ref, bb_ref, (2 * kt2 - 1) * n_blk, tile_n, tile_k)
    simO_ref[...] = jax.lax.dot_general(
        q, cb2[tile_k:2 * tile_k, :], (((1,), (1,)), ((), ())),
        preferred_element_type=jnp.float32)
    _fold(simE_ref, bv_ref, bb_ref, (2 * kt2) * n_blk, tile_n, tile_k)

    @pl.when(kt2 == kt2_num - 1)
    def _finish():
        _fold(simO_ref, bv_ref, bb_ref, (2 * kt2 + 1) * n_blk, tile_n,
              tile_k)
        bv = bv_ref[...]
        k_idx = bb_ref[...] * 128 + jax.lax.broadcasted_iota(
            jnp.int32, (tile_n, 128), 1)
        m = jnp.max(bv, axis=1, keepdims=True)
        winners = jnp.where(bv == m, k_idx, k_total)
        out_ref[0, 0] = jnp.min(winners, axis=1, keepdims=True)


def kernel(x, rand_projs, codebook):
    b, n, d = x.shape
    h, _, e = rand_projs.shape
    _, k, _ = codebook.shape
    rows = b * n

    tile_n = min(1024, rows)
    tile_k = min(1024, k // 2)
    nt = rows // tile_n
    kt2_num = k // (2 * tile_k)

    x2 = x.reshape(rows, d)

    cbn = pl.pallas_call(
        _cbnorm_body,
        grid=(h, k // tile_k),
        in_specs=[pl.BlockSpec((1, tile_k, e), lambda j, s: (j, s, 0))],
        out_specs=pl.BlockSpec((1, tile_k, e), lambda j, s: (j, s, 0)),
        out_shape=jax.ShapeDtypeStruct((h, k, e), jnp.bfloat16),
    )(codebook)

    body = functools.partial(_vq_body, kt2_num=kt2_num, tile_k=tile_k,
                             k_total=k)

    out = pl.pallas_call(
        body,
        grid=(nt, h, kt2_num),
        in_specs=[
            pl.BlockSpec((tile_n, d), lambda i, j, s: (i, 0)),
            pl.BlockSpec((1, d, e), lambda i, j, s: (j, 0, 0)),
            pl.BlockSpec((1, 2 * tile_k, e), lambda i, j, s: (j, s, 0)),
        ],
        out_specs=pl.BlockSpec((1, 1, tile_n, 1),
                               lambda i, j, s: (i, j, 0, 0)),
        out_shape=jax.ShapeDtypeStruct((nt, h, tile_n, 1), jnp.int32),
        scratch_shapes=[
            pltpu.VMEM((tile_n, d), jnp.bfloat16),
            pltpu.VMEM((tile_n, e), jnp.bfloat16),
            pltpu.VMEM((tile_n, 128), jnp.float32),
            pltpu.VMEM((tile_n, 128), jnp.int32),
            pltpu.VMEM((tile_n, tile_k), jnp.float32),
            pltpu.VMEM((tile_n, tile_k), jnp.float32),
        ],
    )(x2, rand_projs, cbn)

    return out.reshape(nt, h, tile_n).transpose(0, 2, 1).reshape(b, n, h)


# speedup vs baseline: 1.6597x; 1.0322x over previous
import functools

import jax
import jax.numpy as jnp
from jax.experimental import pallas as pl
from jax.experimental.pallas import tpu as pltpu

_LN_EPS = 1e-5
_NORM_EPS = 1e-12
_ROW_CHUNK = 128


def _cbnorm_body(cb_ref, out_ref):
    cb = cb_ref[0]
    norm = jnp.sqrt(jnp.sum(cb * cb, axis=1, keepdims=True))
    out_ref[0] = (cb / (norm + _NORM_EPS)).astype(jnp.bfloat16)


def _fold(src, bv_ref, bb_ref, base_blk, tile_n, tile_k):
    n_blk = tile_k // 128
    for r in range(0, tile_n, _ROW_CHUNK):
        bv = bv_ref[r:r + _ROW_CHUNK, :]
        bb = bb_ref[r:r + _ROW_CHUNK, :]
        for cc in range(n_blk):
            seg = src[r:r + _ROW_CHUNK, cc * 128:(cc + 1) * 128]
            mask = seg > bv
            bv = jnp.where(mask, seg, bv)
            bb = jnp.where(mask, base_blk + cc, bb)
        bv_ref[r:r + _ROW_CHUNK, :] = bv
        bb_ref[r:r + _ROW_CHUNK, :] = bb


def _vq_body(x_ref, rp_ref, cbn_ref, out_ref, xn_ref, q_ref, bv_ref,
             bb_ref, simE_ref, simO_ref, *, kt2_num, tile_k, k_total):
    h = pl.program_id(1)
    kt2 = pl.program_id(2)
    tile_n = bv_ref.shape[0]
    n_blk = tile_k // 128

    @pl.when(jnp.logical_and(h == 0, kt2 == 0))
    def _layernorm():
        xt = x_ref[...]
        mean = jnp.mean(xt, axis=1, keepdims=True)
        cent = xt - mean
        var = jnp.mean(cent * cent, axis=1, keepdims=True)
        xn_ref[...] = (cent / jnp.sqrt(var + _LN_EPS)).astype(jnp.bfloat16)

    @pl.when(kt2 == 0)
    def _project():
        proj = jax.lax.dot_general(
            xn_ref[...], rp_ref[0].astype(jnp.bfloat16),
            (((1,), (0,)), ((), ())),
            preferred_element_type=jnp.float32)
        norm = jnp.sqrt(jnp.sum(proj * proj, axis=1, keepdims=True))
        q_ref[...] = (proj / (norm + _NORM_EPS)).astype(jnp.bfloat16)
        bv_ref[...] = jnp.full((tile_n, 128), -jnp.inf, jnp.float32)
        simO_ref[...] = jnp.full((tile_n, tile_k), -jnp.inf, jnp.float32)

    q = q_ref[...]
    cb2 = cbn_ref[0]

    simE_ref[...] = jax.lax.dot_general(
        q, cb2[0:tile_k, :], (((1,), (1,)), ((), ())),
        preferred_element_type=jnp.float32)
    _fold(simO_ref, bv_ref, bb_ref, (2 * kt2 - 1) * n_blk, tile_n, tile_k)
    simO_ref[...] = jax.lax.dot_general(
        q, cb2[tile_k:2 * tile_k, :], (((1,), (1,)), ((), ())),
        preferred_element_type=jnp.float32)
    _fold(simE_ref, bv_ref, bb_ref, (2 * kt2) * n_blk, tile_n, tile_k)

    @pl.when(kt2 == kt2_num - 1)
    def _finish():
        _fold(simO_ref, bv_ref, bb_ref, (2 * kt2 + 1) * n_blk, tile_n,
              tile_k)
        bv = bv_ref[...]
        k_idx = bb_ref[...] * 128 + jax.lax.broadcasted_iota(
            jnp.int32, (tile_n, 128), 1)
        m = jnp.max(bv, axis=1, keepdims=True)
        winners = jnp.where(bv == m, k_idx, k_total)
        out_ref[0, 0] = jnp.min(winners, axis=1, keepdims=True)


def kernel(x, rand_projs, codebook):
    b, n, d = x.shape
    h, _, e = rand_projs.shape
    _, k, _ = codebook.shape
    rows = b * n

    tile_n = min(1024, rows)
    tile_k = min(2048, k // 2)
    nt = rows // tile_n
    kt2_num = k // (2 * tile_k)

    x2 = x.reshape(rows, d)

    cbn = pl.pallas_call(
        _cbnorm_body,
        grid=(h, k // tile_k),
        in_specs=[pl.BlockSpec((1, tile_k, e), lambda j, s: (j, s, 0))],
        out_specs=pl.BlockSpec((1, tile_k, e), lambda j, s: (j, s, 0)),
        out_shape=jax.ShapeDtypeStruct((h, k, e), jnp.bfloat16),
    )(codebook)

    body = functools.partial(_vq_body, kt2_num=kt2_num, tile_k=tile_k,
                             k_total=k)

    out = pl.pallas_call(
        body,
        grid=(nt, h, kt2_num),
        in_specs=[
            pl.BlockSpec((tile_n, d), lambda i, j, s: (i, 0)),
            pl.BlockSpec((1, d, e), lambda i, j, s: (j, 0, 0)),
            pl.BlockSpec((1, 2 * tile_k, e), lambda i, j, s: (j, s, 0)),
        ],
        out_specs=pl.BlockSpec((1, 1, tile_n, 1),
                               lambda i, j, s: (i, j, 0, 0)),
        out_shape=jax.ShapeDtypeStruct((nt, h, tile_n, 1), jnp.int32),
        scratch_shapes=[
            pltpu.VMEM((tile_n, d), jnp.bfloat16),
            pltpu.VMEM((tile_n, e), jnp.bfloat16),
            pltpu.VMEM((tile_n, 128), jnp.float32),
            pltpu.VMEM((tile_n, 128), jnp.int32),
            pltpu.VMEM((tile_n, tile_k), jnp.float32),
            pltpu.VMEM((tile_n, tile_k), jnp.float32),
        ],
    )(x2, rand_projs, cbn)

    return out.reshape(nt, h, tile_n).transpose(0, 2, 1).reshape(b, n, h)


# speedup vs baseline: 2.1025x; 1.2668x over previous
import functools

import jax
import jax.numpy as jnp
from jax.experimental import pallas as pl
from jax.experimental.pallas import tpu as pltpu

_LN_EPS = 1e-5
_NORM_EPS = 1e-12
_ROW_CHUNK = 128
_NSPLIT = 4


def _cbnorm_body(cb_ref, out_ref):
    cb = cb_ref[0]
    norm = jnp.sqrt(jnp.sum(cb * cb, axis=1, keepdims=True))
    out_ref[0] = (cb / (norm + _NORM_EPS)).astype(jnp.bfloat16)


def _fold(src, bv_ref, bb_ref, base_blk, tile_n, tile_k, init):
    n_blk = tile_k // 128
    for r in range(0, tile_n, _ROW_CHUNK):
        if init:
            bv = jnp.full((_ROW_CHUNK, 128), -jnp.inf, jnp.float32)
            bb = jnp.zeros((_ROW_CHUNK, 128), jnp.int32)
        else:
            bv = bv_ref[r:r + _ROW_CHUNK, :]
            bb = bb_ref[r:r + _ROW_CHUNK, :]
        for cc in range(n_blk):
            seg = src[r:r + _ROW_CHUNK, cc * 128:(cc + 1) * 128]
            mask = seg > bv
            bv = jnp.where(mask, seg, bv)
            bb = jnp.where(mask, base_blk + cc, bb)
        bv_ref[r:r + _ROW_CHUNK, :] = bv
        bb_ref[r:r + _ROW_CHUNK, :] = bb


def _vq_body(x_ref, rp_ref, cbn_ref, out_ref, xn_ref, bv_ref, bb_ref,
             *sim_refs, tile_k, k_total):
    h = pl.program_id(1)
    tile_n = x_ref.shape[0]
    n_blk = tile_k // 128

    @pl.when(h == 0)
    def _layernorm():
        xt = x_ref[...]
        mean = jnp.mean(xt, axis=1, keepdims=True)
        cent = xt - mean
        var = jnp.mean(cent * cent, axis=1, keepdims=True)
        xn_ref[...] = (cent / jnp.sqrt(var + _LN_EPS)).astype(jnp.bfloat16)

    proj = jax.lax.dot_general(
        xn_ref[...], rp_ref[0].astype(jnp.bfloat16),
        (((1,), (0,)), ((), ())),
        preferred_element_type=jnp.float32)
    norm = jnp.sqrt(jnp.sum(proj * proj, axis=1, keepdims=True))
    q = (proj / (norm + _NORM_EPS)).astype(jnp.bfloat16)

    cb_all = cbn_ref[0]

    def dot(i):
        sim_refs[i][...] = jax.lax.dot_general(
            q, cb_all[i * tile_k:(i + 1) * tile_k, :],
            (((1,), (1,)), ((), ())),
            preferred_element_type=jnp.float32)

    dot(0)
    dot(1)
    _fold(sim_refs[0], bv_ref, bb_ref, 0, tile_n, tile_k, init=True)
    dot(2)
    _fold(sim_refs[1], bv_ref, bb_ref, n_blk, tile_n, tile_k, init=False)
    dot(3)
    _fold(sim_refs[2], bv_ref, bb_ref, 2 * n_blk, tile_n, tile_k,
          init=False)
    _fold(sim_refs[3], bv_ref, bb_ref, 3 * n_blk, tile_n, tile_k,
          init=False)

    bv = bv_ref[...]
    k_idx = bb_ref[...] * 128 + jax.lax.broadcasted_iota(
        jnp.int32, (tile_n, 128), 1)
    m = jnp.max(bv, axis=1, keepdims=True)
    winners = jnp.where(bv == m, k_idx, k_total)
    out_ref[0, 0] = jnp.min(winners, axis=1, keepdims=True)


def kernel(x, rand_projs, codebook):
    b, n, d = x.shape
    h, _, e = rand_projs.shape
    _, k, _ = codebook.shape
    rows = b * n

    tile_n = min(1024, rows)
    tile_k = k // _NSPLIT
    nt = rows // tile_n

    x2 = x.reshape(rows, d)

    cb_blk = min(2048, k)
    cbn = pl.pallas_call(
        _cbnorm_body,
        grid=(h, k // cb_blk),
        in_specs=[pl.BlockSpec((1, cb_blk, e), lambda j, s: (j, s, 0))],
        out_specs=pl.BlockSpec((1, cb_blk, e), lambda j, s: (j, s, 0)),
        out_shape=jax.ShapeDtypeStruct((h, k, e), jnp.bfloat16),
    )(codebook)

    body = functools.partial(_vq_body, tile_k=tile_k, k_total=k)

    out = pl.pallas_call(
        body,
        grid=(nt, h),
        in_specs=[
            pl.BlockSpec((tile_n, d), lambda i, j: (i, 0)),
            pl.BlockSpec((1, d, e), lambda i, j: (j, 0, 0)),
            pl.BlockSpec((1, k, e), lambda i, j: (j, 0, 0)),
        ],
        out_specs=pl.BlockSpec((1, 1, tile_n, 1), lambda i, j: (i, j, 0, 0)),
        out_shape=jax.ShapeDtypeStruct((nt, h, tile_n, 1), jnp.int32),
        scratch_shapes=[
            pltpu.VMEM((tile_n, d), jnp.bfloat16),
            pltpu.VMEM((tile_n, 128), jnp.float32),
            pltpu.VMEM((tile_n, 128), jnp.int32),
        ] + [pltpu.VMEM((tile_n, tile_k), jnp.float32)
             for _ in range(_NSPLIT)],
    )(x2, rand_projs, cbn)

    return out.reshape(nt, h, tile_n).transpose(0, 2, 1).reshape(b, n, h)


# speedup vs baseline: 2.1042x; 1.0008x over previous
import functools

import jax
import jax.numpy as jnp
from jax.experimental import pallas as pl
from jax.experimental.pallas import tpu as pltpu

_LN_EPS = 1e-5
_NORM_EPS = 1e-12
_ROW_CHUNK = 128
_NSPLIT = 4


def _cbnorm_body(cb_ref, out_ref):
    cb = cb_ref[0]
    norm = jnp.sqrt(jnp.sum(cb * cb, axis=1, keepdims=True))
    out_ref[0] = (cb / (norm + _NORM_EPS)).astype(jnp.bfloat16)


def _vq_body(x_ref, rp_ref, cbn_ref, out_ref, xn_ref, *sim_refs,
             tile_k, k_total):
    h = pl.program_id(1)
    tile_n = x_ref.shape[0]
    n_blk = tile_k // 128

    @pl.when(h == 0)
    def _layernorm():
        xt = x_ref[...]
        mean = jnp.mean(xt, axis=1, keepdims=True)
        cent = xt - mean
        var = jnp.mean(cent * cent, axis=1, keepdims=True)
        xn_ref[...] = (cent / jnp.sqrt(var + _LN_EPS)).astype(jnp.bfloat16)

    proj = jax.lax.dot_general(
        xn_ref[...], rp_ref[0].astype(jnp.bfloat16),
        (((1,), (0,)), ((), ())),
        preferred_element_type=jnp.float32)
    norm = jnp.sqrt(jnp.sum(proj * proj, axis=1, keepdims=True))
    q = (proj / (norm + _NORM_EPS)).astype(jnp.bfloat16)

    cb_all = cbn_ref[0]

    for i in range(_NSPLIT):
        sim_refs[i][...] = jax.lax.dot_general(
            q, cb_all[i * tile_k:(i + 1) * tile_k, :],
            (((1,), (1,)), ((), ())),
            preferred_element_type=jnp.float32)

    lane = jax.lax.broadcasted_iota(jnp.int32, (_ROW_CHUNK, 128), 1)
    for r in range(0, tile_n, _ROW_CHUNK):
        bv = jnp.full((_ROW_CHUNK, 128), -jnp.inf, jnp.float32)
        bb = jnp.zeros((_ROW_CHUNK, 128), jnp.int32)
        for i in range(_NSPLIT):
            src = sim_refs[i]
            for cc in range(n_blk):
                seg = src[r:r + _ROW_CHUNK, cc * 128:(cc + 1) * 128]
                mask = seg > bv
                bv = jnp.where(mask, seg, bv)
                bb = jnp.where(mask, i * n_blk + cc, bb)
        k_idx = bb * 128 + lane
        m = jnp.max(bv, axis=1, keepdims=True)
        winners = jnp.where(bv == m, k_idx, k_total)
        out_ref[0, 0, r:r + _ROW_CHUNK, :] = jnp.min(winners, axis=1,
                                                     keepdims=True)


def kernel(x, rand_projs, codebook):
    b, n, d = x.shape
    h, _, e = rand_projs.shape
    _, k, _ = codebook.shape
    rows = b * n

    tile_n = min(1024, rows)
    tile_k = k // _NSPLIT
    nt = rows // tile_n

    x2 = x.reshape(rows, d)

    cb_blk = min(2048, k)
    cbn = pl.pallas_call(
        _cbnorm_body,
        grid=(h, k // cb_blk),
        in_specs=[pl.BlockSpec((1, cb_blk, e), lambda j, s: (j, s, 0))],
        out_specs=pl.BlockSpec((1, cb_blk, e), lambda j, s: (j, s, 0)),
        out_shape=jax.ShapeDtypeStruct((h, k, e), jnp.bfloat16),
    )(codebook)

    body = functools.partial(_vq_body, tile_k=tile_k, k_total=k)

    out = pl.pallas_call(
        body,
        grid=(nt, h),
        in_specs=[
            pl.BlockSpec((tile_n, d), lambda i, j: (i, 0)),
            pl.BlockSpec((1, d, e), lambda i, j: (j, 0, 0)),
            pl.BlockSpec((1, k, e), lambda i, j: (j, 0, 0)),
        ],
        out_specs=pl.BlockSpec((1, 1, tile_n, 1), lambda i, j: (i, j, 0, 0)),
        out_shape=jax.ShapeDtypeStruct((nt, h, tile_n, 1), jnp.int32),
        scratch_shapes=[
            pltpu.VMEM((tile_n, d), jnp.bfloat16),
        ] + [pltpu.VMEM((tile_n, tile_k), jnp.float32)
             for _ in range(_NSPLIT)],
    )(x2, rand_projs, cbn)

    return out.reshape(nt, h, tile_n).transpose(0, 2, 1).reshape(b, n, h)
